# baseline (speedup 1.0000x reference)
import functools

import jax
import jax.numpy as jnp
from jax import lax
from jax.experimental import pallas as pl
from jax.experimental.pallas import tpu as pltpu
from jax.experimental.pallas import tpu_sc as plsc

N = 100000
NODE_DIM = 128
B = 256
NW = 32
TILE = 128
ACC_ROWS = B + 8
BP_ROWS = 800
SPLIT = 256
SC_Q, SC_R = divmod(SPLIT, NW)
MAX_TPW = SC_Q + 1
TC_R0 = SPLIT * TILE
CHUNK = 2048
TC_STEPS = -(-(N - TC_R0) // CHUNK)


def _sc_segment_sum(x, bp):
    mesh = plsc.VectorSubcoreMesh(core_axis_name="c", subcore_axis_name="s")

    @functools.partial(
        pl.kernel,
        out_type=jax.ShapeDtypeStruct((2, B, NODE_DIM), jnp.float32),
        mesh=mesh,
        scratch_types=[
            pltpu.VMEM((24, TILE), jnp.int32),
            pltpu.VMEM((2, TILE, NODE_DIM), jnp.float32),
            pltpu.VMEM((16, NODE_DIM), jnp.float32),
            pltpu.VMEM_SHARED((ACC_ROWS, NODE_DIM), jnp.float32),
            pltpu.SemaphoreType.DMA,
            pltpu.SemaphoreType.DMA,
        ],
    )
    def k(x_hbm, bp_hbm, sums_out, idx_v, buf, zrow_v, sums_sh, sem0, sem1):
        c = lax.axis_index("c")
        s = lax.axis_index("s")
        wid = c * 16 + s

        zero16 = jnp.zeros((16,), jnp.float32)
        for i in range(16):
            for j in range(NODE_DIM // 16):
                zrow_v[i, pl.ds(j * 16, 16)] = zero16

        pltpu.sync_copy(zrow_v, sums_sh.at[pl.ds(s * 16, 16)])

        @pl.when(s == 0)
        def _():
            pltpu.sync_copy(zrow_v.at[pl.ds(0, 8)], sums_sh.at[pl.ds(B, 8)])

        start = SC_Q * wid + jnp.minimum(wid, SC_R)
        nt = jnp.where(wid < SC_R, SC_Q + 1, SC_Q)
        astart = (start // 8) * 8
        off = start - astart

        pltpu.sync_copy(bp_hbm.at[pl.ds(astart, 24)], idx_v)
        plsc.subcore_barrier()

        sems = (sem0, sem1)

        def issue(i, slot, sem):
            pltpu.async_copy(x_hbm.at[pl.ds((start + i) * TILE, TILE)],
                             buf.at[slot], sem)

        def wait(slot, sem):
            pltpu.make_async_copy(x_hbm.at[pl.ds(0, TILE)],
                                  buf.at[slot], sem).wait()

        @pl.when(nt > 0)
        def _():
            issue(0, 0, sems[0])

        for i in range(MAX_TPW):
            if i + 1 < MAX_TPW:
                @pl.when(i + 1 < nt)
                def _(i=i):
                    issue(i + 1, (i + 1) % 2, sems[(i + 1) % 2])

            @pl.when(i < nt)
            def _(i=i):
                wait(i % 2, sems[i % 2])
                pltpu.sync_copy(buf.at[i % 2], sums_sh.at[idx_v.at[off + i]],
                                add=True)

        plsc.subcore_barrier()

        @pl.when(s == 0)
        def _():
            pltpu.sync_copy(sums_sh.at[pl.ds(0, B)], sums_out.at[c])

    return k(x, bp)


CNT_CHUNK = 32


def _tc_count_body(bp_ref, cnt_ref):
    steps = BP_ROWS // CNT_CHUNK
    width = CNT_CHUNK * TILE
    iota16 = jax.lax.broadcasted_iota(jnp.int32, (16, width), 0).astype(jnp.bfloat16)
    cnt16 = jnp.zeros((16, 16), jnp.float32)
    for k in range(steps):
        blk = bp_ref[pl.ds(k * CNT_CHUNK, CNT_CHUNK), :]
        flat = blk.reshape(1, width)
        hi = (flat >> 4).astype(jnp.bfloat16)
        lo = (flat & 15).astype(jnp.bfloat16)
        eh = (iota16 == hi).astype(jnp.bfloat16)
        el = (iota16 == lo).astype(jnp.bfloat16)
        cnt16 = cnt16 + jax.lax.dot_general(
            eh, el, (((1,), (1,)), ((), ())),
            preferred_element_type=jnp.float32)
    cnt_ref[...] = cnt16


def _tc_segsum_body(x_ref, batch_ref, out_ref):
    i = pl.program_id(0)
    remaining = (N - TC_R0) - i * CHUNK
    ids = jax.lax.broadcasted_iota(jnp.int32, (B, CHUNK), 0)
    cols = jax.lax.broadcasted_iota(jnp.int32, (B, CHUNK), 1)
    bvals = batch_ref[...].reshape(1, CHUNK)
    eb = ((ids == bvals) & (cols < remaining)).astype(jnp.bfloat16)
    rows = jax.lax.broadcasted_iota(jnp.int32, (CHUNK, NODE_DIM), 0)
    xb = jnp.where(rows < remaining, x_ref[...], 0.0).astype(jnp.bfloat16)
    part = jnp.dot(eb, xb, preferred_element_type=jnp.float32)

    @pl.when(i == 0)
    def _():
        out_ref[...] = jnp.zeros_like(out_ref)

    out_ref[...] += part


def _tc_finish_body(sums_ref, tc_ref, cnt_ref, u_ref, W1_ref, b1_ref,
                    gamma_ref, beta_ref, W2_ref, b2_ref, out_ref):
    sums = sums_ref[0] + sums_ref[1] + tc_ref[...]
    c16 = cnt_ref[...]
    bi = jax.lax.broadcasted_iota(jnp.int32, (B, 16), 0)
    ki = jax.lax.broadcasted_iota(jnp.int32, (B, 16), 1)
    hsel = ((bi >> 4) == ki).astype(jnp.float32)
    lsel = ((bi & 15) == ki).astype(jnp.float32)
    tmp = jnp.dot(hsel, c16, preferred_element_type=jnp.float32)
    cnt = jnp.sum(tmp * lsel, axis=1, keepdims=True)
    mean = sums / jnp.clip(cnt, 1.0, None)
    cat = jnp.concatenate([u_ref[...], mean], axis=1)
    h = jnp.dot(cat, W1_ref[...], preferred_element_type=jnp.float32)
    h = h + b1_ref[...][None, :]
    mu = jnp.mean(h, axis=-1, keepdims=True)
    var = jnp.mean((h - mu) ** 2, axis=-1, keepdims=True)
    h = (h - mu) / jnp.sqrt(var + 1e-5) * gamma_ref[...][None, :]
    h = h + beta_ref[...][None, :]
    h = jnp.maximum(h, 0.0)
    y = jnp.dot(h, W2_ref[...], preferred_element_type=jnp.float32)
    out_ref[...] = y + b2_ref[...][None, :]


def kernel(x, edge_index, edge_attr, u, batch, W1, b1, gamma, beta, W2, b2):
    del edge_index, edge_attr
    bp = jnp.pad(batch, (0, BP_ROWS * TILE - N),
                 constant_values=B).reshape(BP_ROWS, TILE)
    cnt = pl.pallas_call(
        _tc_count_body,
        out_shape=jax.ShapeDtypeStruct((16, 16), jnp.float32),
    )(bp)
    tc_part = pl.pallas_call(
        _tc_segsum_body,
        grid=(TC_STEPS,),
        in_specs=[
            pl.BlockSpec((CHUNK, NODE_DIM), lambda i: (TC_R0 // CHUNK + i, 0)),
            pl.BlockSpec((CHUNK,), lambda i: (TC_R0 // CHUNK + i,)),
        ],
        out_specs=pl.BlockSpec((B, NODE_DIM), lambda i: (0, 0)),
        out_shape=jax.ShapeDtypeStruct((B, NODE_DIM), jnp.float32),
    )(x, batch)
    sums2 = _sc_segment_sum(x, bp)
    out = pl.pallas_call(
        _tc_finish_body,
        out_shape=jax.ShapeDtypeStruct((B, W2.shape[1]), jnp.float32),
    )(sums2, tc_part, cnt, u, W1, b1, gamma, beta, W2, b2)
    return out


# speedup vs baseline: 1.1986x; 1.1986x over previous
import functools

import jax
import jax.numpy as jnp
from jax import lax
from jax.experimental import pallas as pl
from jax.experimental.pallas import tpu as pltpu
from jax.experimental.pallas import tpu_sc as plsc

N = 100000
NODE_DIM = 128
B = 256
NW = 32
TILE = 128
ACC_ROWS = B + 8
BP_ROWS = 800
SPLIT = 448
SC_Q, SC_R = divmod(SPLIT, NW)
MAX_TPW = SC_Q + 1
TC_R0 = SPLIT * TILE
CHUNK = 2048
TC_STEPS = -(-(N - TC_R0) // CHUNK)


def _sc_segment_sum(x, bp):
    mesh = plsc.VectorSubcoreMesh(core_axis_name="c", subcore_axis_name="s")

    @functools.partial(
        pl.kernel,
        out_type=jax.ShapeDtypeStruct((2, B, NODE_DIM), jnp.float32),
        mesh=mesh,
        scratch_types=[
            pltpu.VMEM((24, TILE), jnp.int32),
            pltpu.VMEM((2, TILE, NODE_DIM), jnp.float32),
            pltpu.VMEM((16, NODE_DIM), jnp.float32),
            pltpu.VMEM_SHARED((ACC_ROWS, NODE_DIM), jnp.float32),
            pltpu.SemaphoreType.DMA,
            pltpu.SemaphoreType.DMA,
        ],
    )
    def k(x_hbm, bp_hbm, sums_out, idx_v, buf, zrow_v, sums_sh, sem0, sem1):
        c = lax.axis_index("c")
        s = lax.axis_index("s")
        wid = c * 16 + s

        zero16 = jnp.zeros((16,), jnp.float32)
        for i in range(16):
            for j in range(NODE_DIM // 16):
                zrow_v[i, pl.ds(j * 16, 16)] = zero16

        pltpu.sync_copy(zrow_v, sums_sh.at[pl.ds(s * 16, 16)])

        @pl.when(s == 0)
        def _():
            pltpu.sync_copy(zrow_v.at[pl.ds(0, 8)], sums_sh.at[pl.ds(B, 8)])

        start = SC_Q * wid + jnp.minimum(wid, SC_R)
        nt = jnp.where(wid < SC_R, SC_Q + 1, SC_Q)
        astart = (start // 8) * 8
        off = start - astart

        pltpu.sync_copy(bp_hbm.at[pl.ds(astart, 24)], idx_v)
        plsc.subcore_barrier()

        sems = (sem0, sem1)

        def issue(i, slot, sem):
            pltpu.async_copy(x_hbm.at[pl.ds((start + i) * TILE, TILE)],
                             buf.at[slot], sem)

        def wait(slot, sem):
            pltpu.make_async_copy(x_hbm.at[pl.ds(0, TILE)],
                                  buf.at[slot], sem).wait()

        @pl.when(nt > 0)
        def _():
            issue(0, 0, sems[0])

        for i in range(MAX_TPW):
            if i + 1 < MAX_TPW:
                @pl.when(i + 1 < nt)
                def _(i=i):
                    issue(i + 1, (i + 1) % 2, sems[(i + 1) % 2])

            @pl.when(i < nt)
            def _(i=i):
                wait(i % 2, sems[i % 2])
                pltpu.sync_copy(buf.at[i % 2], sums_sh.at[idx_v.at[off + i]],
                                add=True)

        plsc.subcore_barrier()

        @pl.when(s == 0)
        def _():
            pltpu.sync_copy(sums_sh.at[pl.ds(0, B)], sums_out.at[c])

    return k(x, bp)


CNT_CHUNK = 32


def _tc_count_body(bp_ref, cnt_ref):
    steps = BP_ROWS // CNT_CHUNK
    width = CNT_CHUNK * TILE
    iota16 = jax.lax.broadcasted_iota(jnp.int32, (16, width), 0).astype(jnp.bfloat16)
    cnt16 = jnp.zeros((16, 16), jnp.float32)
    for k in range(steps):
        blk = bp_ref[pl.ds(k * CNT_CHUNK, CNT_CHUNK), :]
        flat = blk.reshape(1, width)
        hi = (flat >> 4).astype(jnp.bfloat16)
        lo = (flat & 15).astype(jnp.bfloat16)
        eh = (iota16 == hi).astype(jnp.bfloat16)
        el = (iota16 == lo).astype(jnp.bfloat16)
        cnt16 = cnt16 + jax.lax.dot_general(
            eh, el, (((1,), (1,)), ((), ())),
            preferred_element_type=jnp.float32)
    cnt_ref[...] = cnt16


def _tc_segsum_body(x_ref, batch_ref, out_ref):
    i = pl.program_id(0)
    ids = jax.lax.broadcasted_iota(jnp.int32, (B, CHUNK), 0).astype(jnp.bfloat16)
    bvals = batch_ref[...].reshape(1, CHUNK).astype(jnp.bfloat16)
    eq = ids == bvals
    remaining = (N - TC_R0) - i * CHUNK
    cols = jax.lax.broadcasted_iota(jnp.int32, (B, CHUNK), 1)
    eq = eq & (cols < remaining)
    eb = eq.astype(jnp.bfloat16)
    xb = x_ref[...].astype(jnp.bfloat16)
    if (N - TC_R0) % CHUNK != 0:
        rows = jax.lax.broadcasted_iota(jnp.int32, (CHUNK, NODE_DIM), 0)
        xb = jnp.where(rows < (N - TC_R0) - i * CHUNK, xb, jnp.bfloat16(0))
    part = jnp.dot(eb, xb, preferred_element_type=jnp.float32)

    @pl.when(i == 0)
    def _():
        out_ref[...] = jnp.zeros_like(out_ref)

    out_ref[...] += part


def _tc_finish_body(sums_ref, tc_ref, cnt_ref, u_ref, W1_ref, b1_ref,
                    gamma_ref, beta_ref, W2_ref, b2_ref, out_ref):
    sums = sums_ref[0] + sums_ref[1] + tc_ref[...]
    c16 = cnt_ref[...]
    bi = jax.lax.broadcasted_iota(jnp.int32, (B, 16), 0)
    ki = jax.lax.broadcasted_iota(jnp.int32, (B, 16), 1)
    hsel = ((bi >> 4) == ki).astype(jnp.float32)
    lsel = ((bi & 15) == ki).astype(jnp.float32)
    tmp = jnp.dot(hsel, c16, preferred_element_type=jnp.float32)
    cnt = jnp.sum(tmp * lsel, axis=1, keepdims=True)
    mean = sums / jnp.clip(cnt, 1.0, None)
    cat = jnp.concatenate([u_ref[...], mean], axis=1)
    h = jnp.dot(cat, W1_ref[...], preferred_element_type=jnp.float32)
    h = h + b1_ref[...][None, :]
    mu = jnp.mean(h, axis=-1, keepdims=True)
    var = jnp.mean((h - mu) ** 2, axis=-1, keepdims=True)
    h = (h - mu) / jnp.sqrt(var + 1e-5) * gamma_ref[...][None, :]
    h = h + beta_ref[...][None, :]
    h = jnp.maximum(h, 0.0)
    y = jnp.dot(h, W2_ref[...], preferred_element_type=jnp.float32)
    out_ref[...] = y + b2_ref[...][None, :]


def kernel(x, edge_index, edge_attr, u, batch, W1, b1, gamma, beta, W2, b2):
    del edge_index, edge_attr
    bp = jnp.pad(batch, (0, BP_ROWS * TILE - N),
                 constant_values=B).reshape(BP_ROWS, TILE)
    cnt = pl.pallas_call(
        _tc_count_body,
        out_shape=jax.ShapeDtypeStruct((16, 16), jnp.float32),
    )(bp)
    tc_part = pl.pallas_call(
        _tc_segsum_body,
        grid=(TC_STEPS,),
        in_specs=[
            pl.BlockSpec((CHUNK, NODE_DIM), lambda i: (TC_R0 // CHUNK + i, 0)),
            pl.BlockSpec((CHUNK,), lambda i: (TC_R0 // CHUNK + i,)),
        ],
        out_specs=pl.BlockSpec((B, NODE_DIM), lambda i: (0, 0)),
        out_shape=jax.ShapeDtypeStruct((B, NODE_DIM), jnp.float32),
    )(x, batch)
    sums2 = _sc_segment_sum(x, bp)
    out = pl.pallas_call(
        _tc_finish_body,
        out_shape=jax.ShapeDtypeStruct((B, W2.shape[1]), jnp.float32),
    )(sums2, tc_part, cnt, u, W1, b1, gamma, beta, W2, b2)
    return out


# speedup vs baseline: 1.3060x; 1.0896x over previous
import functools

import jax
import jax.numpy as jnp
from jax import lax
from jax.experimental import pallas as pl
from jax.experimental.pallas import tpu as pltpu
from jax.experimental.pallas import tpu_sc as plsc

N = 100000
NODE_DIM = 128
B = 256
NW = 32
TILE = 128
ACC_ROWS = B + 8
BP_ROWS = 800
SPLIT = 480
SC_Q, SC_R = divmod(SPLIT, NW)
MAX_TPW = SC_Q + 1
TC_R0 = SPLIT * TILE
CHUNK = 4096
TC_STEPS = -(-(N - TC_R0) // CHUNK)


def _sc_segment_sum(x, bp):
    mesh = plsc.VectorSubcoreMesh(core_axis_name="c", subcore_axis_name="s")

    @functools.partial(
        pl.kernel,
        out_type=jax.ShapeDtypeStruct((2, B, NODE_DIM), jnp.float32),
        mesh=mesh,
        scratch_types=[
            pltpu.VMEM((24, TILE), jnp.int32),
            pltpu.VMEM((2, TILE, NODE_DIM), jnp.float32),
            pltpu.VMEM((16, NODE_DIM), jnp.float32),
            pltpu.VMEM_SHARED((ACC_ROWS, NODE_DIM), jnp.float32),
            pltpu.SemaphoreType.DMA,
            pltpu.SemaphoreType.DMA,
        ],
    )
    def k(x_hbm, bp_hbm, sums_out, idx_v, buf, zrow_v, sums_sh, sem0, sem1):
        c = lax.axis_index("c")
        s = lax.axis_index("s")
        wid = c * 16 + s

        zero16 = jnp.zeros((16,), jnp.float32)
        for i in range(16):
            for j in range(NODE_DIM // 16):
                zrow_v[i, pl.ds(j * 16, 16)] = zero16

        pltpu.sync_copy(zrow_v, sums_sh.at[pl.ds(s * 16, 16)])

        @pl.when(s == 0)
        def _():
            pltpu.sync_copy(zrow_v.at[pl.ds(0, 8)], sums_sh.at[pl.ds(B, 8)])

        start = SC_Q * wid + jnp.minimum(wid, SC_R)
        nt = jnp.where(wid < SC_R, SC_Q + 1, SC_Q)
        astart = (start // 8) * 8
        off = start - astart

        pltpu.sync_copy(bp_hbm.at[pl.ds(astart, 24)], idx_v)
        plsc.subcore_barrier()

        sems = (sem0, sem1)

        def issue(i, slot, sem):
            pltpu.async_copy(x_hbm.at[pl.ds((start + i) * TILE, TILE)],
                             buf.at[slot], sem)

        def wait(slot, sem):
            pltpu.make_async_copy(x_hbm.at[pl.ds(0, TILE)],
                                  buf.at[slot], sem).wait()

        @pl.when(nt > 0)
        def _():
            issue(0, 0, sems[0])

        for i in range(MAX_TPW):
            if i + 1 < MAX_TPW:
                @pl.when(i + 1 < nt)
                def _(i=i):
                    issue(i + 1, (i + 1) % 2, sems[(i + 1) % 2])

            @pl.when(i < nt)
            def _(i=i):
                wait(i % 2, sems[i % 2])
                pltpu.sync_copy(buf.at[i % 2], sums_sh.at[idx_v.at[off + i]],
                                add=True)

        plsc.subcore_barrier()

        @pl.when(s == 0)
        def _():
            pltpu.sync_copy(sums_sh.at[pl.ds(0, B)], sums_out.at[c])

    return k(x, bp)


CNT_CHUNK = 32


def _tc_count_body(bp_ref, cnt_ref):
    steps = BP_ROWS // CNT_CHUNK
    width = CNT_CHUNK * TILE
    iota16 = jax.lax.broadcasted_iota(jnp.int32, (16, width), 0).astype(jnp.bfloat16)
    cnt16 = jnp.zeros((16, 16), jnp.float32)
    for k in range(steps):
        blk = bp_ref[pl.ds(k * CNT_CHUNK, CNT_CHUNK), :]
        flat = blk.reshape(1, width)
        hi = (flat >> 4).astype(jnp.bfloat16)
        lo = (flat & 15).astype(jnp.bfloat16)
        eh = (iota16 == hi).astype(jnp.bfloat16)
        el = (iota16 == lo).astype(jnp.bfloat16)
        cnt16 = cnt16 + jax.lax.dot_general(
            eh, el, (((1,), (1,)), ((), ())),
            preferred_element_type=jnp.float32)
    cnt_ref[...] = cnt16


def _tc_segsum_body(x_ref, batch_ref, out_ref):
    i = pl.program_id(0)
    ids = jax.lax.broadcasted_iota(jnp.int32, (B, CHUNK), 0).astype(jnp.bfloat16)
    bvals = batch_ref[...].reshape(1, CHUNK).astype(jnp.bfloat16)
    eq = ids == bvals
    remaining = (N - TC_R0) - i * CHUNK
    cols = jax.lax.broadcasted_iota(jnp.int32, (B, CHUNK), 1)
    eq = eq & (cols < remaining)
    eb = eq.astype(jnp.bfloat16)
    xb = x_ref[...].astype(jnp.bfloat16)
    if (N - TC_R0) % CHUNK != 0:
        rows = jax.lax.broadcasted_iota(jnp.int32, (CHUNK, NODE_DIM), 0)
        xb = jnp.where(rows < (N - TC_R0) - i * CHUNK, xb, jnp.bfloat16(0))
    part = jnp.dot(eb, xb, preferred_element_type=jnp.float32)

    @pl.when(i == 0)
    def _():
        out_ref[...] = jnp.zeros_like(out_ref)

    out_ref[...] += part


def _tc_finish_body(sums_ref, tc_ref, cnt_ref, u_ref, W1_ref, b1_ref,
                    gamma_ref, beta_ref, W2_ref, b2_ref, out_ref):
    sums = sums_ref[0] + sums_ref[1] + tc_ref[...]
    c16 = cnt_ref[...]
    bi = jax.lax.broadcasted_iota(jnp.int32, (B, 16), 0)
    ki = jax.lax.broadcasted_iota(jnp.int32, (B, 16), 1)
    hsel = ((bi >> 4) == ki).astype(jnp.float32)
    lsel = ((bi & 15) == ki).astype(jnp.float32)
    tmp = jnp.dot(hsel, c16, preferred_element_type=jnp.float32)
    cnt = jnp.sum(tmp * lsel, axis=1, keepdims=True)
    mean = sums / jnp.clip(cnt, 1.0, None)
    cat = jnp.concatenate([u_ref[...], mean], axis=1)
    h = jnp.dot(cat, W1_ref[...], preferred_element_type=jnp.float32)
    h = h + b1_ref[...][None, :]
    mu = jnp.mean(h, axis=-1, keepdims=True)
    var = jnp.mean((h - mu) ** 2, axis=-1, keepdims=True)
    h = (h - mu) / jnp.sqrt(var + 1e-5) * gamma_ref[...][None, :]
    h = h + beta_ref[...][None, :]
    h = jnp.maximum(h, 0.0)
    y = jnp.dot(h, W2_ref[...], preferred_element_type=jnp.float32)
    out_ref[...] = y + b2_ref[...][None, :]


def kernel(x, edge_index, edge_attr, u, batch, W1, b1, gamma, beta, W2, b2):
    del edge_index, edge_attr
    bp = jnp.pad(batch, (0, BP_ROWS * TILE - N),
                 constant_values=B).reshape(BP_ROWS, TILE)
    cnt = pl.pallas_call(
        _tc_count_body,
        out_shape=jax.ShapeDtypeStruct((16, 16), jnp.float32),
    )(bp)
    tc_part = pl.pallas_call(
        _tc_segsum_body,
        grid=(TC_STEPS,),
        in_specs=[
            pl.BlockSpec((CHUNK, NODE_DIM), lambda i: (TC_R0 // CHUNK + i, 0)),
            pl.BlockSpec((CHUNK,), lambda i: (TC_R0 // CHUNK + i,)),
        ],
        out_specs=pl.BlockSpec((B, NODE_DIM), lambda i: (0, 0)),
        out_shape=jax.ShapeDtypeStruct((B, NODE_DIM), jnp.float32),
    )(x, batch)
    sums2 = _sc_segment_sum(x, bp)
    out = pl.pallas_call(
        _tc_finish_body,
        out_shape=jax.ShapeDtypeStruct((B, W2.shape[1]), jnp.float32),
    )(sums2, tc_part, cnt, u, W1, b1, gamma, beta, W2, b2)
    return out


# speedup vs baseline: 1.3066x; 1.0005x over previous
import functools

import jax
import jax.numpy as jnp
from jax import lax
from jax.experimental import pallas as pl
from jax.experimental.pallas import tpu as pltpu
from jax.experimental.pallas import tpu_sc as plsc

N = 100000
NODE_DIM = 128
B = 256
NW = 32
TILE = 128
ACC_ROWS = B + 8
BP_ROWS = 800
SPLIT = 480
SC_Q, SC_R = divmod(SPLIT, NW)
MAX_TPW = SC_Q + 1
TC_R0 = SPLIT * TILE
CHUNK = 4096
TC_STEPS = -(-(N - TC_R0) // CHUNK)


def _sc_segment_sum(x, bp):
    mesh = plsc.VectorSubcoreMesh(core_axis_name="c", subcore_axis_name="s")

    @functools.partial(
        pl.kernel,
        out_type=jax.ShapeDtypeStruct((2, B, NODE_DIM), jnp.float32),
        mesh=mesh,
        scratch_types=[
            pltpu.VMEM((24, TILE), jnp.int32),
            pltpu.VMEM((2, TILE, NODE_DIM), jnp.float32),
            pltpu.VMEM((16, NODE_DIM), jnp.float32),
            pltpu.VMEM_SHARED((ACC_ROWS, NODE_DIM), jnp.float32),
            pltpu.SemaphoreType.DMA,
            pltpu.SemaphoreType.DMA,
        ],
    )
    def k(x_hbm, bp_hbm, sums_out, idx_v, buf, zrow_v, sums_sh, sem0, sem1):
        c = lax.axis_index("c")
        s = lax.axis_index("s")
        wid = c * 16 + s

        zero16 = jnp.zeros((16,), jnp.float32)
        for i in range(16):
            for j in range(NODE_DIM // 16):
                zrow_v[i, pl.ds(j * 16, 16)] = zero16

        pltpu.sync_copy(zrow_v, sums_sh.at[pl.ds(s * 16, 16)])

        @pl.when(s == 0)
        def _():
            pltpu.sync_copy(zrow_v.at[pl.ds(0, 8)], sums_sh.at[pl.ds(B, 8)])

        start = SC_Q * wid + jnp.minimum(wid, SC_R)
        nt = jnp.where(wid < SC_R, SC_Q + 1, SC_Q)
        astart = (start // 8) * 8
        off = start - astart

        pltpu.sync_copy(bp_hbm.at[pl.ds(astart, 24)], idx_v)
        plsc.subcore_barrier()

        sems = (sem0, sem1)

        def issue(i, slot, sem):
            pltpu.async_copy(x_hbm.at[pl.ds((start + i) * TILE, TILE)],
                             buf.at[slot], sem)

        def wait(slot, sem):
            pltpu.make_async_copy(x_hbm.at[pl.ds(0, TILE)],
                                  buf.at[slot], sem).wait()

        @pl.when(nt > 0)
        def _():
            issue(0, 0, sems[0])

        for i in range(MAX_TPW):
            if i + 1 < MAX_TPW:
                @pl.when(i + 1 < nt)
                def _(i=i):
                    issue(i + 1, (i + 1) % 2, sems[(i + 1) % 2])

            @pl.when(i < nt)
            def _(i=i):
                wait(i % 2, sems[i % 2])
                pltpu.sync_copy(buf.at[i % 2], sums_sh.at[idx_v.at[off + i]],
                                add=True)

        plsc.subcore_barrier()

        @pl.when(s == 0)
        def _():
            pltpu.sync_copy(sums_sh.at[pl.ds(0, B)], sums_out.at[c])

    return k(x, bp)


BPC = BP_ROWS // TC_STEPS
assert BP_ROWS % TC_STEPS == 0 and BPC % 16 == 0


def _tc_segsum_body(x_ref, batch_ref, bp_ref, out_ref, cnt_ref):
    i = pl.program_id(0)
    ids = jax.lax.broadcasted_iota(jnp.int32, (B, CHUNK), 0).astype(jnp.bfloat16)
    bvals = batch_ref[...].reshape(1, CHUNK).astype(jnp.bfloat16)
    eq = ids == bvals
    remaining = (N - TC_R0) - i * CHUNK
    cols = jax.lax.broadcasted_iota(jnp.int32, (B, CHUNK), 1)
    eq = eq & (cols < remaining)
    eb = eq.astype(jnp.bfloat16)
    xb = x_ref[...].astype(jnp.bfloat16)
    if (N - TC_R0) % CHUNK != 0:
        rows = jax.lax.broadcasted_iota(jnp.int32, (CHUNK, NODE_DIM), 0)
        xb = jnp.where(rows < (N - TC_R0) - i * CHUNK, xb, jnp.bfloat16(0))
    part = jnp.dot(eb, xb, preferred_element_type=jnp.float32)

    iota16 = jax.lax.broadcasted_iota(jnp.int32, (16, 16 * TILE), 0).astype(jnp.bfloat16)
    c16 = jnp.zeros((16, 16), jnp.float32)
    for k in range(BPC // 16):
        blk = bp_ref[pl.ds(k * 16, 16), :]
        flat = blk.reshape(1, 16 * TILE)
        hi = (flat >> 4).astype(jnp.bfloat16)
        lo = (flat & 15).astype(jnp.bfloat16)
        eh = (iota16 == hi).astype(jnp.bfloat16)
        el = (iota16 == lo).astype(jnp.bfloat16)
        c16 = c16 + jax.lax.dot_general(
            eh, el, (((1,), (1,)), ((), ())),
            preferred_element_type=jnp.float32)

    @pl.when(i == 0)
    def _():
        out_ref[...] = jnp.zeros_like(out_ref)
        cnt_ref[...] = jnp.zeros_like(cnt_ref)

    out_ref[...] += part
    cnt_ref[...] += c16


def _tc_finish_body(sums_ref, tc_ref, cnt_ref, u_ref, W1_ref, b1_ref,
                    gamma_ref, beta_ref, W2_ref, b2_ref, out_ref):
    sums = sums_ref[0] + sums_ref[1] + tc_ref[...]
    c16 = cnt_ref[...]
    bi = jax.lax.broadcasted_iota(jnp.int32, (B, 16), 0)
    ki = jax.lax.broadcasted_iota(jnp.int32, (B, 16), 1)
    hsel = ((bi >> 4) == ki).astype(jnp.float32)
    lsel = ((bi & 15) == ki).astype(jnp.float32)
    tmp = jnp.dot(hsel, c16, preferred_element_type=jnp.float32)
    cnt = jnp.sum(tmp * lsel, axis=1, keepdims=True)
    mean = sums / jnp.clip(cnt, 1.0, None)
    cat = jnp.concatenate([u_ref[...], mean], axis=1)
    h = jnp.dot(cat, W1_ref[...], preferred_element_type=jnp.float32)
    h = h + b1_ref[...][None, :]
    mu = jnp.mean(h, axis=-1, keepdims=True)
    var = jnp.mean((h - mu) ** 2, axis=-1, keepdims=True)
    h = (h - mu) / jnp.sqrt(var + 1e-5) * gamma_ref[...][None, :]
    h = h + beta_ref[...][None, :]
    h = jnp.maximum(h, 0.0)
    y = jnp.dot(h, W2_ref[...], preferred_element_type=jnp.float32)
    out_ref[...] = y + b2_ref[...][None, :]


def kernel(x, edge_index, edge_attr, u, batch, W1, b1, gamma, beta, W2, b2):
    del edge_index, edge_attr
    bp = jnp.pad(batch, (0, BP_ROWS * TILE - N),
                 constant_values=B).reshape(BP_ROWS, TILE)
    tc_part, cnt = pl.pallas_call(
        _tc_segsum_body,
        grid=(TC_STEPS,),
        in_specs=[
            pl.BlockSpec((CHUNK, NODE_DIM), lambda i: (TC_R0 // CHUNK + i, 0)),
            pl.BlockSpec((CHUNK,), lambda i: (TC_R0 // CHUNK + i,)),
            pl.BlockSpec((BPC, TILE), lambda i: (i, 0)),
        ],
        out_specs=[
            pl.BlockSpec((B, NODE_DIM), lambda i: (0, 0)),
            pl.BlockSpec((16, 16), lambda i: (0, 0)),
        ],
        out_shape=[
            jax.ShapeDtypeStruct((B, NODE_DIM), jnp.float32),
            jax.ShapeDtypeStruct((16, 16), jnp.float32),
        ],
    )(x, batch, bp)
    sums2 = _sc_segment_sum(x, bp)
    out = pl.pallas_call(
        _tc_finish_body,
        out_shape=jax.ShapeDtypeStruct((B, W2.shape[1]), jnp.float32),
    )(sums2, tc_part, cnt, u, W1, b1, gamma, beta, W2, b2)
    return out


# speedup vs baseline: 1.3319x; 1.0193x over previous
import functools

import jax
import jax.numpy as jnp
from jax import lax
from jax.experimental import pallas as pl
from jax.experimental.pallas import tpu as pltpu
from jax.experimental.pallas import tpu_sc as plsc

N = 100000
NODE_DIM = 128
B = 256
NW = 32
TILE = 128
ACC_ROWS = B + 8
SPLIT = 448
SC_Q, SC_R = divmod(SPLIT, NW)
MAX_TPW = SC_Q + (1 if SC_R else 0)
TC_R0 = SPLIT * TILE
CHUNK = 4096
TC_STEPS = -(-(N - TC_R0) // CHUNK)
BPC = 16 * (-(-(N // TILE + 1) // (16 * TC_STEPS)))
BP_ROWS = BPC * TC_STEPS


def _sc_segment_sum(x, bp):
    mesh = plsc.VectorSubcoreMesh(core_axis_name="c", subcore_axis_name="s")

    @functools.partial(
        pl.kernel,
        out_type=jax.ShapeDtypeStruct((2, B, NODE_DIM), jnp.float32),
        mesh=mesh,
        scratch_types=[
            pltpu.VMEM((24, TILE), jnp.int32),
            pltpu.VMEM((2, TILE, NODE_DIM), jnp.float32),
            pltpu.VMEM((16, NODE_DIM), jnp.float32),
            pltpu.VMEM_SHARED((ACC_ROWS, NODE_DIM), jnp.float32),
            pltpu.SemaphoreType.DMA,
            pltpu.SemaphoreType.DMA,
        ],
    )
    def k(x_hbm, bp_hbm, sums_out, idx_v, buf, zrow_v, sums_sh, sem0, sem1):
        c = lax.axis_index("c")
        s = lax.axis_index("s")
        wid = c * 16 + s

        zero16 = jnp.zeros((16,), jnp.float32)
        for i in range(16):
            for j in range(NODE_DIM // 16):
                zrow_v[i, pl.ds(j * 16, 16)] = zero16

        pltpu.sync_copy(zrow_v, sums_sh.at[pl.ds(s * 16, 16)])

        @pl.when(s == 0)
        def _():
            pltpu.sync_copy(zrow_v.at[pl.ds(0, 8)], sums_sh.at[pl.ds(B, 8)])

        start = SC_Q * wid + jnp.minimum(wid, SC_R)
        nt = jnp.where(wid < SC_R, SC_Q + 1, SC_Q)
        astart = (start // 8) * 8
        off = start - astart

        pltpu.sync_copy(bp_hbm.at[pl.ds(astart, 24)], idx_v)
        plsc.subcore_barrier()

        sems = (sem0, sem1)

        def issue(i, slot, sem):
            pltpu.async_copy(x_hbm.at[pl.ds((start + i) * TILE, TILE)],
                             buf.at[slot], sem)

        def wait(slot, sem):
            pltpu.make_async_copy(x_hbm.at[pl.ds(0, TILE)],
                                  buf.at[slot], sem).wait()

        @pl.when(nt > 0)
        def _():
            issue(0, 0, sems[0])

        for i in range(MAX_TPW):
            if i + 1 < MAX_TPW:
                @pl.when(i + 1 < nt)
                def _(i=i):
                    issue(i + 1, (i + 1) % 2, sems[(i + 1) % 2])

            @pl.when(i < nt)
            def _(i=i):
                wait(i % 2, sems[i % 2])
                pltpu.sync_copy(buf.at[i % 2], sums_sh.at[idx_v.at[off + i]],
                                add=True)

        plsc.subcore_barrier()

        @pl.when(s == 0)
        def _():
            pltpu.sync_copy(sums_sh.at[pl.ds(0, B)], sums_out.at[c])

    return k(x, bp)


def _tc_segsum_body(x_ref, batch_ref, bp_ref, out_ref, cnt_ref):
    i = pl.program_id(0)
    ids = jax.lax.broadcasted_iota(jnp.int32, (B, CHUNK), 0).astype(jnp.bfloat16)
    bvals = batch_ref[...].reshape(1, CHUNK).astype(jnp.bfloat16)
    eq = ids == bvals
    remaining = (N - TC_R0) - i * CHUNK
    cols = jax.lax.broadcasted_iota(jnp.int32, (B, CHUNK), 1)
    eq = eq & (cols < remaining)
    eb = eq.astype(jnp.bfloat16)
    xb = x_ref[...].astype(jnp.bfloat16)
    if (N - TC_R0) % CHUNK != 0:
        rows = jax.lax.broadcasted_iota(jnp.int32, (CHUNK, NODE_DIM), 0)
        xb = jnp.where(rows < (N - TC_R0) - i * CHUNK, xb, jnp.bfloat16(0))
    part = jnp.dot(eb, xb, preferred_element_type=jnp.float32)

    iota16 = jax.lax.broadcasted_iota(jnp.int32, (16, 16 * TILE), 0).astype(jnp.bfloat16)
    c16 = jnp.zeros((16, 16), jnp.float32)
    for k in range(BPC // 16):
        blk = bp_ref[pl.ds(k * 16, 16), :]
        flat = blk.reshape(1, 16 * TILE)
        hi = (flat >> 4).astype(jnp.bfloat16)
        lo = (flat & 15).astype(jnp.bfloat16)
        eh = (iota16 == hi).astype(jnp.bfloat16)
        el = (iota16 == lo).astype(jnp.bfloat16)
        c16 = c16 + jax.lax.dot_general(
            eh, el, (((1,), (1,)), ((), ())),
            preferred_element_type=jnp.float32)

    @pl.when(i == 0)
    def _():
        out_ref[...] = jnp.zeros_like(out_ref)
        cnt_ref[...] = jnp.zeros_like(cnt_ref)

    out_ref[...] += part
    cnt_ref[...] += c16


def _tc_finish_body(sums_ref, tc_ref, cnt_ref, u_ref, W1_ref, b1_ref,
                    gamma_ref, beta_ref, W2_ref, b2_ref, out_ref):
    sums = sums_ref[0] + sums_ref[1] + tc_ref[...]
    c16 = cnt_ref[...]
    bi = jax.lax.broadcasted_iota(jnp.int32, (B, 16), 0)
    ki = jax.lax.broadcasted_iota(jnp.int32, (B, 16), 1)
    hsel = ((bi >> 4) == ki).astype(jnp.float32)
    lsel = ((bi & 15) == ki).astype(jnp.float32)
    tmp = jnp.dot(hsel, c16, preferred_element_type=jnp.float32)
    cnt = jnp.sum(tmp * lsel, axis=1, keepdims=True)
    mean = sums / jnp.clip(cnt, 1.0, None)
    cat = jnp.concatenate([u_ref[...], mean], axis=1)
    h = jnp.dot(cat, W1_ref[...], preferred_element_type=jnp.float32)
    h = h + b1_ref[...][None, :]
    mu = jnp.mean(h, axis=-1, keepdims=True)
    var = jnp.mean((h - mu) ** 2, axis=-1, keepdims=True)
    h = (h - mu) / jnp.sqrt(var + 1e-5) * gamma_ref[...][None, :]
    h = h + beta_ref[...][None, :]
    h = jnp.maximum(h, 0.0)
    y = jnp.dot(h, W2_ref[...], preferred_element_type=jnp.float32)
    out_ref[...] = y + b2_ref[...][None, :]


def kernel(x, edge_index, edge_attr, u, batch, W1, b1, gamma, beta, W2, b2):
    del edge_index, edge_attr
    bp = jnp.pad(batch, (0, BP_ROWS * TILE - N),
                 constant_values=B).reshape(BP_ROWS, TILE)
    tc_part, cnt = pl.pallas_call(
        _tc_segsum_body,
        grid=(TC_STEPS,),
        in_specs=[
            pl.BlockSpec((CHUNK, NODE_DIM), lambda i: (TC_R0 // CHUNK + i, 0)),
            pl.BlockSpec((CHUNK,), lambda i: (TC_R0 // CHUNK + i,)),
            pl.BlockSpec((BPC, TILE), lambda i: (i, 0)),
        ],
        out_specs=[
            pl.BlockSpec((B, NODE_DIM), lambda i: (0, 0)),
            pl.BlockSpec((16, 16), lambda i: (0, 0)),
        ],
        out_shape=[
            jax.ShapeDtypeStruct((B, NODE_DIM), jnp.float32),
            jax.ShapeDtypeStruct((16, 16), jnp.float32),
        ],
    )(x, batch, bp)
    sums2 = _sc_segment_sum(x, bp)
    out = pl.pallas_call(
        _tc_finish_body,
        out_shape=jax.ShapeDtypeStruct((B, W2.shape[1]), jnp.float32),
    )(sums2, tc_part, cnt, u, W1, b1, gamma, beta, W2, b2)
    return out
